# baseline (speedup 1.0000x reference)
import dataclasses
import functools

import jax
import jax.numpy as jnp
from jax import lax
from jax.experimental import pallas as pl
from jax.experimental.pallas import tpu as pltpu
from jax.experimental.pallas import tpu_sc as plsc

_NC = 2
_NS = 16
_NW = _NC * _NS
_L = 16

_R = 200
_C = 16384
_C_W = _C // _NW
_BC = 128
_NBLK = _C_W // _BC


def _make_sc_kernel():
    mesh = plsc.VectorSubcoreMesh(core_axis_name="c", subcore_axis_name="s")

    cp = pltpu.CompilerParams()
    if "needs_layout_passes" in pltpu.CompilerParams.__dataclass_fields__:
        cp = dataclasses.replace(cp, needs_layout_passes=False)

    @functools.partial(
        pl.kernel,
        mesh=mesh,
        out_type=jax.ShapeDtypeStruct((_R, _C), jnp.int32),
        scratch_types=[
            pltpu.VMEM((128,), jnp.int32),
            pltpu.VMEM((2, _R, _BC), jnp.int32),
            pltpu.VMEM((2, _R, _BC), jnp.int32),
            pltpu.SemaphoreType.DMA((2,)),
            pltpu.SemaphoreType.DMA((2,)),
        ],
        compiler_params=cp,
    )
    def sc_kernel(in_hbm, map_hbm, out_hbm, table_v, in_b, out_b, sin, sout):
        wid = lax.axis_index("s") * _NC + lax.axis_index("c")
        base = wid * _C_W

        def in_copy(blk, par):
            return pltpu.make_async_copy(
                in_hbm.at[:, pl.ds(base + blk * _BC, _BC)],
                in_b.at[par], sin.at[par])

        def out_copy(blk, par):
            return pltpu.make_async_copy(
                out_b.at[par], out_hbm.at[:, pl.ds(base + blk * _BC, _BC)],
                sout.at[par])

        in_copy(0, 0).start()
        pltpu.sync_copy(map_hbm, table_v)

        def do_blk(blk, _):
            par = lax.rem(blk, 2)
            in_copy(blk, par).wait()

            @pl.when(blk + 1 < _NBLK)
            def _():
                in_copy(blk + 1, 1 - par).start()

            @pl.when(blk >= 2)
            def _():
                out_copy(blk - 2, par).wait()

            @plsc.parallel_loop(0, _R, step=1, unroll=8)
            def translate(r):
                for g in range(_BC // _L):
                    s = pl.ds(g * _L, _L)
                    out_b[par, r, s] = plsc.load_gather(
                        table_v, [in_b[par, r, s]])

            out_copy(blk, par).start()
            return 0

        lax.fori_loop(0, _NBLK, do_blk, 0)
        out_copy(_NBLK - 2, (_NBLK - 2) % 2).wait()
        out_copy(_NBLK - 1, (_NBLK - 1) % 2).wait()

    return sc_kernel


_sc_kernel = _make_sc_kernel()


@jax.jit
def kernel(inputs, mapping):
    out_t = _sc_kernel(inputs.T, mapping)
    return out_t.T
